# baseline (speedup 1.0000x reference)
import functools

import jax
import jax.numpy as jnp
from jax import lax
from jax.experimental import pallas as pl
from jax.experimental.pallas import tpu as pltpu
from jax.experimental.pallas import tpu_sc as plsc

N_NODES = 10000
N_EDGES = 320000
D_FEAT = 128
L1 = 64
L2 = 32

NC = 2
NS = 16
NW = NC * NS
N_PAD = 10240
N_PER_W = N_EDGES // NW
PADT = 240
W_EDGES = N_PER_W + PADT
_DEG_W = 16

_SC_PARAMS = pltpu.CompilerParams(use_tc_tiling_on_sc=False)


def _mesh():
    return plsc.VectorSubcoreMesh(core_axis_name="c", subcore_axis_name="s",
                                  num_cores=NC, num_subcores=NS)


def _stage_indices(ei_hbm, row, idx_v, base):
    pltpu.sync_copy(ei_hbm.at[row].at[pl.ds(base, N_PER_W)],
                    idx_v.at[pl.ds(0, N_PER_W)])
    for t in range(PADT // 16):
        idx_v[pl.ds(N_PER_W + 16 * t, 16)] = (
            lax.iota(jnp.int32, 16) + (N_NODES + 16 * t))


def _make_agg(d, chunk, nbuf):
    kb = W_EDGES // chunk
    stripe = N_PAD // NS

    @functools.partial(
        pl.kernel,
        out_type=jax.ShapeDtypeStruct((NC, N_PAD, d), jnp.float32),
        mesh=_mesh(),
        scratch_types=[
            pltpu.VMEM((W_EDGES,), jnp.int32),
            pltpu.VMEM((W_EDGES,), jnp.int32),
            pltpu.VMEM((nbuf, chunk, d), jnp.float32),
            pltpu.VMEM_SHARED((N_PAD, d), jnp.float32),
            pltpu.SemaphoreType.DMA((nbuf,)),
        ],
        compiler_params=_SC_PARAMS,
    )
    def agg(table_hbm, ei_hbm, zeros_hbm, out_hbm,
            src_v, dst_v, rows_v, acc_sh, sems):
        c = lax.axis_index("c")
        s = lax.axis_index("s")
        base = (c * NS + s) * N_PER_W
        _stage_indices(ei_hbm, 0, src_v, base)
        _stage_indices(ei_hbm, 1, dst_v, base)
        pltpu.sync_copy(zeros_hbm.at[pl.ds(s * stripe, stripe)],
                        acc_sh.at[pl.ds(s * stripe, stripe)])
        plsc.subcore_barrier()

        def start_gather(j, b):
            pltpu.async_copy(
                table_hbm.at[src_v.at[pl.ds(j * chunk, chunk)]],
                rows_v.at[b], sems.at[b])

        for b in range(nbuf - 1):
            start_gather(b, b)

        def body(j, carry):
            b = lax.rem(j, nbuf)
            f = j + nbuf - 1

            @pl.when(f < kb)
            def _():
                start_gather(f, lax.rem(f, nbuf))

            pltpu.make_async_copy(
                table_hbm.at[src_v.at[pl.ds(j * chunk, chunk)]],
                rows_v.at[b], sems.at[b]).wait()
            pltpu.sync_copy(rows_v.at[b],
                            acc_sh.at[dst_v.at[pl.ds(j * chunk, chunk)]],
                            add=True)
            return carry

        lax.fori_loop(0, kb, body, 0)
        plsc.subcore_barrier()
        pltpu.sync_copy(acc_sh.at[pl.ds(s * stripe, stripe)],
                        out_hbm.at[c].at[pl.ds(s * stripe, stripe)])

    return agg


def _make_deg(chunk):
    kb = W_EDGES // chunk
    stripe = N_PAD // NS

    @functools.partial(
        pl.kernel,
        out_type=jax.ShapeDtypeStruct((NC, N_PAD, _DEG_W), jnp.float32),
        mesh=_mesh(),
        scratch_types=[
            pltpu.VMEM((W_EDGES,), jnp.int32),
            pltpu.VMEM((chunk, _DEG_W), jnp.float32),
            pltpu.VMEM_SHARED((N_PAD, _DEG_W), jnp.float32),
            pltpu.SemaphoreType.DMA,
        ],
        compiler_params=_SC_PARAMS,
    )
    def deg(ei_hbm, ones_hbm, zeros_hbm, out_hbm, dst_v, ones_v, acc_sh, sem):
        c = lax.axis_index("c")
        s = lax.axis_index("s")
        base = (c * NS + s) * N_PER_W
        _stage_indices(ei_hbm, 1, dst_v, base)
        pltpu.sync_copy(ones_hbm, ones_v)
        pltpu.sync_copy(zeros_hbm.at[pl.ds(s * stripe, stripe)],
                        acc_sh.at[pl.ds(s * stripe, stripe)])
        plsc.subcore_barrier()

        def fire(j, carry):
            pltpu.async_copy(
                ones_v, acc_sh.at[dst_v.at[pl.ds(j * chunk, chunk)]],
                sem, add=True)
            return carry

        def drain(j, carry):
            pltpu.make_async_copy(
                ones_v, acc_sh.at[dst_v.at[pl.ds(j * chunk, chunk)]],
                sem).wait()
            return carry

        lax.fori_loop(0, kb, fire, 0)
        lax.fori_loop(0, kb, drain, 0)
        plsc.subcore_barrier()
        pltpu.sync_copy(acc_sh.at[pl.ds(s * stripe, stripe)],
                        out_hbm.at[c].at[pl.ds(s * stripe, stripe)])

    return deg


@functools.lru_cache(maxsize=None)
def _agg_fn(d, chunk, nbuf):
    return _make_agg(d, chunk, nbuf)


@functools.lru_cache(maxsize=None)
def _deg_fn(chunk):
    return _make_deg(chunk)


def _agg_deg(*a):
    return _deg_fn(1024)(*a)


def _agg64(*a):
    return _agg_fn(L1, 512, 2)(*a)


def _agg32(*a):
    return _agg_fn(L2, 1024, 2)(*a)


_BLK = 2048
_GRID = N_PAD // _BLK


def _mm1_body(x_ref, w_ref, out_ref):
    out_ref[...] = jnp.dot(x_ref[...], w_ref[...],
                           preferred_element_type=jnp.float32)


def _mm_relu_body(a_ref, b_ref, w_ref, out_ref):
    a = jnp.maximum(a_ref[...] + b_ref[...], 0.0)
    out_ref[...] = jnp.dot(a, w_ref[...], preferred_element_type=jnp.float32)


def _mm_head_body(a_ref, b_ref, w_ref, b3_ref, out_ref):
    a = jnp.maximum(a_ref[...] + b_ref[...], 0.0)
    out_ref[...] = jnp.dot(a, w_ref[...],
                           preferred_element_type=jnp.float32) + b3_ref[...]


def _tc_matmul(x, w):
    din, dout = x.shape[1], w.shape[1]
    return pl.pallas_call(
        _mm1_body,
        grid=(_GRID,),
        in_specs=[
            pl.BlockSpec((_BLK, din), lambda i: (i, 0)),
            pl.BlockSpec((din, dout), lambda i: (0, 0)),
        ],
        out_specs=pl.BlockSpec((_BLK, dout), lambda i: (i, 0)),
        out_shape=jax.ShapeDtypeStruct((N_PAD, dout), jnp.float32),
    )(x, w)


def _tc_mm_relu(a, b, w):
    din, dout = a.shape[1], w.shape[1]
    return pl.pallas_call(
        _mm_relu_body,
        grid=(_GRID,),
        in_specs=[
            pl.BlockSpec((_BLK, din), lambda i: (i, 0)),
            pl.BlockSpec((1, din), lambda i: (0, 0)),
            pl.BlockSpec((din, dout), lambda i: (0, 0)),
        ],
        out_specs=pl.BlockSpec((_BLK, dout), lambda i: (i, 0)),
        out_shape=jax.ShapeDtypeStruct((N_PAD, dout), jnp.float32),
    )(a, b, w)


def _tc_mm_head(a, b, w, b3):
    din = a.shape[1]
    return pl.pallas_call(
        _mm_head_body,
        grid=(_GRID,),
        in_specs=[
            pl.BlockSpec((_BLK, din), lambda i: (i, 0)),
            pl.BlockSpec((1, din), lambda i: (0, 0)),
            pl.BlockSpec((din, 1), lambda i: (0, 0)),
            pl.BlockSpec((1, 1), lambda i: (0, 0)),
        ],
        out_specs=pl.BlockSpec((_BLK, 1), lambda i: (i, 0)),
        out_shape=jax.ShapeDtypeStruct((N_PAD, 1), jnp.float32),
    )(a, b, w, b3)


def kernel(x, edge_index, W1, b1, W2, b2, W3, b3):
    ei = edge_index.astype(jnp.int32)

    x_pad = jnp.pad(x, ((0, N_PAD - N_NODES), (0, 0)))
    ones16 = jnp.ones((1024, _DEG_W), jnp.float32)
    zeros16 = jnp.zeros((N_PAD, _DEG_W), jnp.float32)
    zeros64 = jnp.zeros((N_PAD, L1), jnp.float32)
    zeros32 = jnp.zeros((N_PAD, L2), jnp.float32)

    degp = _agg_deg(ei, ones16, zeros16)
    h1 = _tc_matmul(x_pad, W1)

    dinv = lax.rsqrt(degp[0, :, 0] + degp[1, :, 0] + 1.0)[:, None]
    g1 = dinv * h1
    p = _agg64(g1, ei, zeros64)
    a1 = dinv * (p[0] + p[1] + g1)
    g2 = dinv * _tc_mm_relu(a1, b1.reshape(1, L1), W2)
    q = _agg32(g2, ei, zeros32)
    a2 = dinv * (q[0] + q[1] + g2)
    out = _tc_mm_head(a2, b2.reshape(1, L2), W3, b3.reshape(1, 1))
    return out[:N_NODES]


# speedup vs baseline: 1.0144x; 1.0144x over previous
import functools

import jax
import jax.numpy as jnp
from jax import lax
from jax.experimental import pallas as pl
from jax.experimental.pallas import tpu as pltpu
from jax.experimental.pallas import tpu_sc as plsc

N_NODES = 10000
N_EDGES = 320000
D_FEAT = 128
L1 = 64
L2 = 32

NC = 2
NS = 16
NW = NC * NS
N_PAD = 10240
N_PER_W = N_EDGES // NW
PADT = 240
W_EDGES = N_PER_W + PADT
_DEG_W = 16

_SC_PARAMS = pltpu.CompilerParams(use_tc_tiling_on_sc=False)


def _mesh():
    return plsc.VectorSubcoreMesh(core_axis_name="c", subcore_axis_name="s",
                                  num_cores=NC, num_subcores=NS)


def _stage_indices(e_hbm, idx_v, base):
    pltpu.sync_copy(e_hbm.at[pl.ds(base, N_PER_W)],
                    idx_v.at[pl.ds(0, N_PER_W)])
    for t in range(PADT // 16):
        idx_v[pl.ds(N_PER_W + 16 * t, 16)] = (
            lax.iota(jnp.int32, 16) + (N_NODES + 16 * t))


def _make_agg(d, chunk, nbuf):
    kb = W_EDGES // chunk
    stripe = N_PAD // NS

    @functools.partial(
        pl.kernel,
        out_type=jax.ShapeDtypeStruct((NC, N_PAD, d), jnp.float32),
        mesh=_mesh(),
        scratch_types=[
            pltpu.VMEM((W_EDGES,), jnp.int32),
            pltpu.VMEM((W_EDGES,), jnp.int32),
            pltpu.VMEM((nbuf, chunk, d), jnp.float32),
            pltpu.VMEM_SHARED((N_PAD, d), jnp.float32),
            pltpu.SemaphoreType.DMA((nbuf,)),
        ],
        compiler_params=_SC_PARAMS,
    )
    def agg(table_hbm, srcs_hbm, dsts_hbm, zeros_hbm, out_hbm,
            src_v, dst_v, rows_v, acc_sh, sems):
        c = lax.axis_index("c")
        s = lax.axis_index("s")
        base = (c * NS + s) * N_PER_W
        _stage_indices(srcs_hbm, src_v, base)
        _stage_indices(dsts_hbm, dst_v, base)
        pltpu.sync_copy(zeros_hbm.at[pl.ds(s * stripe, stripe)],
                        acc_sh.at[pl.ds(s * stripe, stripe)])
        plsc.subcore_barrier()

        def start_gather(j, b):
            pltpu.async_copy(
                table_hbm.at[src_v.at[pl.ds(j * chunk, chunk)]],
                rows_v.at[b], sems.at[b])

        for b in range(nbuf - 1):
            start_gather(b, b)

        def body(j, carry):
            b = lax.rem(j, nbuf)
            f = j + nbuf - 1

            @pl.when(f < kb)
            def _():
                start_gather(f, lax.rem(f, nbuf))

            pltpu.make_async_copy(
                table_hbm.at[src_v.at[pl.ds(j * chunk, chunk)]],
                rows_v.at[b], sems.at[b]).wait()
            pltpu.sync_copy(rows_v.at[b],
                            acc_sh.at[dst_v.at[pl.ds(j * chunk, chunk)]],
                            add=True)
            return carry

        lax.fori_loop(0, kb, body, 0)
        plsc.subcore_barrier()
        pltpu.sync_copy(acc_sh.at[pl.ds(s * stripe, stripe)],
                        out_hbm.at[c].at[pl.ds(s * stripe, stripe)])

    return agg


def _make_deg(chunk):
    kb = W_EDGES // chunk
    stripe = N_PAD // NS

    @functools.partial(
        pl.kernel,
        out_type=jax.ShapeDtypeStruct((NC, N_PAD, _DEG_W), jnp.float32),
        mesh=_mesh(),
        scratch_types=[
            pltpu.VMEM((W_EDGES,), jnp.int32),
            pltpu.VMEM((chunk, _DEG_W), jnp.float32),
            pltpu.VMEM_SHARED((N_PAD, _DEG_W), jnp.float32),
            pltpu.SemaphoreType.DMA,
        ],
        compiler_params=_SC_PARAMS,
    )
    def deg(dsts_hbm, ones_hbm, zeros_hbm, out_hbm, dst_v, ones_v, acc_sh, sem):
        c = lax.axis_index("c")
        s = lax.axis_index("s")
        base = (c * NS + s) * N_PER_W
        _stage_indices(dsts_hbm, dst_v, base)
        pltpu.sync_copy(ones_hbm, ones_v)
        pltpu.sync_copy(zeros_hbm.at[pl.ds(s * stripe, stripe)],
                        acc_sh.at[pl.ds(s * stripe, stripe)])
        plsc.subcore_barrier()

        def fire(j, carry):
            pltpu.async_copy(
                ones_v, acc_sh.at[dst_v.at[pl.ds(j * chunk, chunk)]],
                sem, add=True)
            return carry

        def drain(j, carry):
            pltpu.make_async_copy(
                ones_v, acc_sh.at[dst_v.at[pl.ds(j * chunk, chunk)]],
                sem).wait()
            return carry

        lax.fori_loop(0, kb, fire, 0)
        lax.fori_loop(0, kb, drain, 0)
        plsc.subcore_barrier()
        pltpu.sync_copy(acc_sh.at[pl.ds(s * stripe, stripe)],
                        out_hbm.at[c].at[pl.ds(s * stripe, stripe)])

    return deg


@functools.lru_cache(maxsize=None)
def _agg_fn(d, chunk, nbuf):
    return _make_agg(d, chunk, nbuf)


@functools.lru_cache(maxsize=None)
def _deg_fn(chunk):
    return _make_deg(chunk)


def _agg_deg(*a):
    return _deg_fn(1024)(*a)


def _agg64(*a):
    return _agg_fn(L1, 512, 2)(*a)


def _agg32(*a):
    return _agg_fn(L2, 1024, 2)(*a)


_BLK = 2048
_GRID = N_PAD // _BLK


def _dinv_of(degp):
    return lax.rsqrt(degp[0] + degp[1] + 1.0)[:, None]


def _mm_body(x_ref, w_ref, out_ref):
    out_ref[...] = jnp.dot(x_ref[...], w_ref[...],
                           preferred_element_type=jnp.float32)


def _scale_body(h_ref, degp_ref, out_ref):
    out_ref[...] = _dinv_of(degp_ref[...]) * h_ref[...]


def _k2_body(p_ref, degp_ref, g1_ref, b_ref, w_ref, out_ref):
    dinv = _dinv_of(degp_ref[...])
    a = jnp.maximum(dinv * (p_ref[0] + p_ref[1] + g1_ref[...]) + b_ref[...],
                    0.0)
    out_ref[...] = dinv * jnp.dot(a, w_ref[...],
                                  preferred_element_type=jnp.float32)


def _k3_body(q_ref, degp_ref, g2_ref, b_ref, w_ref, b3_ref, out_ref):
    dinv = _dinv_of(degp_ref[...])
    a = jnp.maximum(dinv * (q_ref[0] + q_ref[1] + g2_ref[...]) + b_ref[...],
                    0.0)
    out_ref[...] = jnp.dot(a, w_ref[...],
                           preferred_element_type=jnp.float32) + b3_ref[...]


def _tc_matmul(x, w):
    return pl.pallas_call(
        _mm_body,
        grid=(_GRID,),
        in_specs=[
            pl.BlockSpec((_BLK, D_FEAT), lambda i: (i, 0)),
            pl.BlockSpec((D_FEAT, L1), lambda i: (0, 0)),
        ],
        out_specs=pl.BlockSpec((_BLK, L1), lambda i: (i, 0)),
        out_shape=jax.ShapeDtypeStruct((N_PAD, L1), jnp.float32),
    )(x, w)


def _tc_scale(h, degp):
    return pl.pallas_call(
        _scale_body,
        grid=(_GRID,),
        in_specs=[
            pl.BlockSpec((_BLK, L1), lambda i: (i, 0)),
            pl.BlockSpec((NC, _BLK), lambda i: (0, i)),
        ],
        out_specs=pl.BlockSpec((_BLK, L1), lambda i: (i, 0)),
        out_shape=jax.ShapeDtypeStruct((N_PAD, L1), jnp.float32),
    )(h, degp)


def _tc_layer2(p, degp, g1, b1, w2):
    return pl.pallas_call(
        _k2_body,
        grid=(_GRID,),
        in_specs=[
            pl.BlockSpec((NC, _BLK, L1), lambda i: (0, i, 0)),
            pl.BlockSpec((NC, _BLK), lambda i: (0, i)),
            pl.BlockSpec((_BLK, L1), lambda i: (i, 0)),
            pl.BlockSpec((1, L1), lambda i: (0, 0)),
            pl.BlockSpec((L1, L2), lambda i: (0, 0)),
        ],
        out_specs=pl.BlockSpec((_BLK, L2), lambda i: (i, 0)),
        out_shape=jax.ShapeDtypeStruct((N_PAD, L2), jnp.float32),
    )(p, degp, g1, b1, w2)


def _tc_head(q, degp, g2, b2, w3, b3):
    return pl.pallas_call(
        _k3_body,
        grid=(_GRID,),
        in_specs=[
            pl.BlockSpec((NC, _BLK, L2), lambda i: (0, i, 0)),
            pl.BlockSpec((NC, _BLK), lambda i: (0, i)),
            pl.BlockSpec((_BLK, L2), lambda i: (i, 0)),
            pl.BlockSpec((1, L2), lambda i: (0, 0)),
            pl.BlockSpec((L2, 1), lambda i: (0, 0)),
            pl.BlockSpec((1, 1), lambda i: (0, 0)),
        ],
        out_specs=pl.BlockSpec((_BLK, 1), lambda i: (i, 0)),
        out_shape=jax.ShapeDtypeStruct((N_PAD, 1), jnp.float32),
    )(q, degp, g2, b2, w3, b3)


def kernel(x, edge_index, W1, b1, W2, b2, W3, b3):
    ei = edge_index.astype(jnp.int32)
    srcs = ei[0]
    dsts = ei[1]

    x_pad = jnp.pad(x, ((0, N_PAD - N_NODES), (0, 0)))
    ones16 = jnp.ones((1024, _DEG_W), jnp.float32)
    zeros16 = jnp.zeros((N_PAD, _DEG_W), jnp.float32)
    zeros64 = jnp.zeros((N_PAD, L1), jnp.float32)
    zeros32 = jnp.zeros((N_PAD, L2), jnp.float32)

    degp = _agg_deg(dsts, ones16, zeros16)[:, :, 0]
    h1 = _tc_matmul(x_pad, W1)

    g1 = _tc_scale(h1, degp)
    p = _agg64(g1, srcs, dsts, zeros64)
    g2 = _tc_layer2(p, degp, g1, b1.reshape(1, L1), W2)
    q = _agg32(g2, srcs, dsts, zeros32)
    out = _tc_head(q, degp, g2, b2.reshape(1, L2), W3, b3.reshape(1, 1))
    return out[:N_NODES]


# speedup vs baseline: 1.0725x; 1.0572x over previous
import functools

import jax
import jax.numpy as jnp
from jax import lax
from jax.experimental import pallas as pl
from jax.experimental.pallas import tpu as pltpu
from jax.experimental.pallas import tpu_sc as plsc

N_NODES = 10000
N_EDGES = 320000
D_FEAT = 128
L1 = 64
L2 = 32

NC = 2
NS = 16
NW = NC * NS
N_PAD = 10240
N_PER_W = N_EDGES // NW
PADT = 240
W_EDGES = N_PER_W + PADT
_DEG_W = 16

_SC_PARAMS = pltpu.CompilerParams(use_tc_tiling_on_sc=False)


def _mesh():
    return plsc.VectorSubcoreMesh(core_axis_name="c", subcore_axis_name="s",
                                  num_cores=NC, num_subcores=NS)


def _stage_indices(ei_hbm, row, idx_v, base):
    pltpu.sync_copy(ei_hbm.at[row].at[pl.ds(base, N_PER_W)],
                    idx_v.at[pl.ds(0, N_PER_W)])
    for t in range(PADT // 16):
        idx_v[pl.ds(N_PER_W + 16 * t, 16)] = (
            lax.iota(jnp.int32, 16) + (N_NODES + 16 * t))


def _make_agg(d, chunk, nbuf):
    kb = W_EDGES // chunk
    stripe = N_PAD // NS

    @functools.partial(
        pl.kernel,
        out_type=jax.ShapeDtypeStruct((NC, N_PAD, d), jnp.float32),
        mesh=_mesh(),
        scratch_types=[
            pltpu.VMEM((W_EDGES,), jnp.int32),
            pltpu.VMEM((W_EDGES,), jnp.int32),
            pltpu.VMEM((nbuf, chunk, d), jnp.float32),
            pltpu.VMEM_SHARED((N_PAD, d), jnp.float32),
            pltpu.SemaphoreType.DMA((nbuf,)),
        ],
        compiler_params=_SC_PARAMS,
    )
    def agg(table_hbm, ei_hbm, zeros_hbm, out_hbm,
            src_v, dst_v, rows_v, acc_sh, sems):
        c = lax.axis_index("c")
        s = lax.axis_index("s")
        base = (c * NS + s) * N_PER_W
        _stage_indices(ei_hbm, 0, src_v, base)
        _stage_indices(ei_hbm, 1, dst_v, base)
        pltpu.sync_copy(zeros_hbm.at[pl.ds(s * stripe, stripe)],
                        acc_sh.at[pl.ds(s * stripe, stripe)])
        plsc.subcore_barrier()

        def start_gather(j, b):
            pltpu.async_copy(
                table_hbm.at[src_v.at[pl.ds(j * chunk, chunk)]],
                rows_v.at[b], sems.at[b])

        for b in range(nbuf - 1):
            start_gather(b, b)

        def body(j, carry):
            b = lax.rem(j, nbuf)
            f = j + nbuf - 1

            @pl.when(f < kb)
            def _():
                start_gather(f, lax.rem(f, nbuf))

            pltpu.make_async_copy(
                table_hbm.at[src_v.at[pl.ds(j * chunk, chunk)]],
                rows_v.at[b], sems.at[b]).wait()
            pltpu.sync_copy(rows_v.at[b],
                            acc_sh.at[dst_v.at[pl.ds(j * chunk, chunk)]],
                            add=True)
            return carry

        lax.fori_loop(0, kb, body, 0)
        plsc.subcore_barrier()
        pltpu.sync_copy(acc_sh.at[pl.ds(s * stripe, stripe)],
                        out_hbm.at[c].at[pl.ds(s * stripe, stripe)])

    return agg


def _make_deg(chunk):
    kb = W_EDGES // chunk
    stripe = N_PAD // NS

    @functools.partial(
        pl.kernel,
        out_type=jax.ShapeDtypeStruct((NC, N_PAD, _DEG_W), jnp.float32),
        mesh=_mesh(),
        scratch_types=[
            pltpu.VMEM((W_EDGES,), jnp.int32),
            pltpu.VMEM((chunk, _DEG_W), jnp.float32),
            pltpu.VMEM_SHARED((N_PAD, _DEG_W), jnp.float32),
            pltpu.SemaphoreType.DMA,
        ],
        compiler_params=_SC_PARAMS,
    )
    def deg(ei_hbm, ones_hbm, zeros_hbm, out_hbm, dst_v, ones_v, acc_sh, sem):
        c = lax.axis_index("c")
        s = lax.axis_index("s")
        base = (c * NS + s) * N_PER_W
        _stage_indices(ei_hbm, 1, dst_v, base)
        pltpu.sync_copy(ones_hbm, ones_v)
        pltpu.sync_copy(zeros_hbm.at[pl.ds(s * stripe, stripe)],
                        acc_sh.at[pl.ds(s * stripe, stripe)])
        plsc.subcore_barrier()

        def fire(j, carry):
            pltpu.async_copy(
                ones_v, acc_sh.at[dst_v.at[pl.ds(j * chunk, chunk)]],
                sem, add=True)
            return carry

        def drain(j, carry):
            pltpu.make_async_copy(
                ones_v, acc_sh.at[dst_v.at[pl.ds(j * chunk, chunk)]],
                sem).wait()
            return carry

        lax.fori_loop(0, kb, fire, 0)
        lax.fori_loop(0, kb, drain, 0)
        plsc.subcore_barrier()
        pltpu.sync_copy(acc_sh.at[pl.ds(s * stripe, stripe)],
                        out_hbm.at[c].at[pl.ds(s * stripe, stripe)])

    return deg


@functools.lru_cache(maxsize=None)
def _agg_fn(d, chunk, nbuf):
    return _make_agg(d, chunk, nbuf)


@functools.lru_cache(maxsize=None)
def _deg_fn(chunk):
    return _make_deg(chunk)


def _agg_deg(*a):
    return _deg_fn(1024)(*a)


def _agg64(*a):
    return _agg_fn(L1, 512, 2)(*a)


def _agg32(*a):
    return _agg_fn(L2, 1024, 2)(*a)


_BLK = 2048
_GRID = N_PAD // _BLK


def _dinv_of(degp):
    return lax.rsqrt(degp[0] + degp[1] + 1.0)[:, None]


def _mm_body(x_ref, w_ref, out_ref):
    out_ref[...] = jnp.dot(x_ref[...], w_ref[...],
                           preferred_element_type=jnp.float32)


def _scale_body(h_ref, degp_ref, out_ref):
    out_ref[...] = _dinv_of(degp_ref[...]) * h_ref[...]


def _k2_body(p_ref, degp_ref, g1_ref, b_ref, w_ref, out_ref):
    dinv = _dinv_of(degp_ref[...])
    a = jnp.maximum(dinv * (p_ref[0] + p_ref[1] + g1_ref[...]) + b_ref[...],
                    0.0)
    out_ref[...] = dinv * jnp.dot(a, w_ref[...],
                                  preferred_element_type=jnp.float32)


def _k3_body(q_ref, degp_ref, g2_ref, b_ref, w_ref, b3_ref, out_ref):
    dinv = _dinv_of(degp_ref[...])
    a = jnp.maximum(dinv * (q_ref[0] + q_ref[1] + g2_ref[...]) + b_ref[...],
                    0.0)
    out_ref[...] = jnp.dot(a, w_ref[...],
                           preferred_element_type=jnp.float32) + b3_ref[...]


def _tc_matmul(x, w):
    return pl.pallas_call(
        _mm_body,
        grid=(_GRID,),
        in_specs=[
            pl.BlockSpec((_BLK, D_FEAT), lambda i: (i, 0)),
            pl.BlockSpec((D_FEAT, L1), lambda i: (0, 0)),
        ],
        out_specs=pl.BlockSpec((_BLK, L1), lambda i: (i, 0)),
        out_shape=jax.ShapeDtypeStruct((N_PAD, L1), jnp.float32),
    )(x, w)


def _tc_scale(h, degp):
    return pl.pallas_call(
        _scale_body,
        grid=(_GRID,),
        in_specs=[
            pl.BlockSpec((_BLK, L1), lambda i: (i, 0)),
            pl.BlockSpec((NC, _BLK), lambda i: (0, i)),
        ],
        out_specs=pl.BlockSpec((_BLK, L1), lambda i: (i, 0)),
        out_shape=jax.ShapeDtypeStruct((N_PAD, L1), jnp.float32),
    )(h, degp)


def _tc_layer2(p, degp, g1, b1, w2):
    return pl.pallas_call(
        _k2_body,
        grid=(_GRID,),
        in_specs=[
            pl.BlockSpec((NC, _BLK, L1), lambda i: (0, i, 0)),
            pl.BlockSpec((NC, _BLK), lambda i: (0, i)),
            pl.BlockSpec((_BLK, L1), lambda i: (i, 0)),
            pl.BlockSpec((1, L1), lambda i: (0, 0)),
            pl.BlockSpec((L1, L2), lambda i: (0, 0)),
        ],
        out_specs=pl.BlockSpec((_BLK, L2), lambda i: (i, 0)),
        out_shape=jax.ShapeDtypeStruct((N_PAD, L2), jnp.float32),
    )(p, degp, g1, b1, w2)


def _tc_head(q, degp, g2, b2, w3, b3):
    return pl.pallas_call(
        _k3_body,
        grid=(_GRID,),
        in_specs=[
            pl.BlockSpec((NC, _BLK, L2), lambda i: (0, i, 0)),
            pl.BlockSpec((NC, _BLK), lambda i: (0, i)),
            pl.BlockSpec((_BLK, L2), lambda i: (i, 0)),
            pl.BlockSpec((1, L2), lambda i: (0, 0)),
            pl.BlockSpec((L2, 1), lambda i: (0, 0)),
            pl.BlockSpec((1, 1), lambda i: (0, 0)),
        ],
        out_specs=pl.BlockSpec((_BLK, 1), lambda i: (i, 0)),
        out_shape=jax.ShapeDtypeStruct((N_PAD, 1), jnp.float32),
    )(q, degp, g2, b2, w3, b3)


def kernel(x, edge_index, W1, b1, W2, b2, W3, b3):
    ei = edge_index.astype(jnp.int32)

    x_pad = jnp.pad(x, ((0, N_PAD - N_NODES), (0, 0)))
    ones16 = jnp.ones((1024, _DEG_W), jnp.float32)
    zeros16 = jnp.zeros((N_PAD, _DEG_W), jnp.float32)
    zeros64 = jnp.zeros((N_PAD, L1), jnp.float32)
    zeros32 = jnp.zeros((N_PAD, L2), jnp.float32)

    degp = _agg_deg(ei, ones16, zeros16)[:, :, 0]
    h1 = _tc_matmul(x_pad, W1)

    g1 = _tc_scale(h1, degp)
    p = _agg64(g1, ei, zeros64)
    g2 = _tc_layer2(p, degp, g1, b1.reshape(1, L1), W2)
    q = _agg32(g2, ei, zeros32)
    out = _tc_head(q, degp, g2, b2.reshape(1, L2), W3, b3.reshape(1, 1))
    return out[:N_NODES]


# speedup vs baseline: 1.0932x; 1.0193x over previous
import functools

import jax
import jax.numpy as jnp
from jax import lax
from jax.experimental import pallas as pl
from jax.experimental.pallas import tpu as pltpu
from jax.experimental.pallas import tpu_sc as plsc

N_NODES = 10000
N_EDGES = 320000
D_FEAT = 128
L1 = 64
L2 = 32

NC = 2
NS = 16
NW = NC * NS
N_PAD = 10240
N_PER_W = N_EDGES // NW
PADT = 240
W_EDGES = N_PER_W + PADT
_DEG_W = 8

_SC_PARAMS = pltpu.CompilerParams(use_tc_tiling_on_sc=False)


def _mesh():
    return plsc.VectorSubcoreMesh(core_axis_name="c", subcore_axis_name="s",
                                  num_cores=NC, num_subcores=NS)


def _stage_indices(ei_hbm, row, idx_v, base):
    pltpu.sync_copy(ei_hbm.at[row].at[pl.ds(base, N_PER_W)],
                    idx_v.at[pl.ds(0, N_PER_W)])
    for t in range(PADT // 16):
        idx_v[pl.ds(N_PER_W + 16 * t, 16)] = (
            lax.iota(jnp.int32, 16) + (N_NODES + 16 * t))


def _make_agg(d, chunk, nbuf):
    kb = W_EDGES // chunk
    stripe = N_PAD // NS

    @functools.partial(
        pl.kernel,
        out_type=jax.ShapeDtypeStruct((NC, N_PAD, d), jnp.float32),
        mesh=_mesh(),
        scratch_types=[
            pltpu.VMEM((W_EDGES,), jnp.int32),
            pltpu.VMEM((W_EDGES,), jnp.int32),
            pltpu.VMEM((nbuf, chunk, d), jnp.float32),
            pltpu.VMEM_SHARED((N_PAD, d), jnp.float32),
            pltpu.SemaphoreType.DMA((nbuf,)),
        ],
        compiler_params=_SC_PARAMS,
    )
    def agg(table_hbm, ei_hbm, zeros_hbm, out_hbm,
            src_v, dst_v, rows_v, acc_sh, sems):
        c = lax.axis_index("c")
        s = lax.axis_index("s")
        base = (c * NS + s) * N_PER_W
        _stage_indices(ei_hbm, 0, src_v, base)
        _stage_indices(ei_hbm, 1, dst_v, base)
        pltpu.sync_copy(zeros_hbm.at[pl.ds(s * stripe, stripe)],
                        acc_sh.at[pl.ds(s * stripe, stripe)])
        plsc.subcore_barrier()

        def start_gather(j, b):
            pltpu.async_copy(
                table_hbm.at[src_v.at[pl.ds(j * chunk, chunk)]],
                rows_v.at[b], sems.at[b])

        for b in range(nbuf - 1):
            start_gather(b, b)

        def body(j, carry):
            b = lax.rem(j, nbuf)
            f = j + nbuf - 1

            @pl.when(f < kb)
            def _():
                start_gather(f, lax.rem(f, nbuf))

            pltpu.make_async_copy(
                table_hbm.at[src_v.at[pl.ds(j * chunk, chunk)]],
                rows_v.at[b], sems.at[b]).wait()
            pltpu.sync_copy(rows_v.at[b],
                            acc_sh.at[dst_v.at[pl.ds(j * chunk, chunk)]],
                            add=True)
            return carry

        lax.fori_loop(0, kb, body, 0)
        plsc.subcore_barrier()
        pltpu.sync_copy(acc_sh.at[pl.ds(s * stripe, stripe)],
                        out_hbm.at[c].at[pl.ds(s * stripe, stripe)])

    return agg


def _make_deg(chunk):
    kb = W_EDGES // chunk
    stripe = N_PAD // NS

    @functools.partial(
        pl.kernel,
        out_type=jax.ShapeDtypeStruct((NC, N_PAD, _DEG_W), jnp.float32),
        mesh=_mesh(),
        scratch_types=[
            pltpu.VMEM((W_EDGES,), jnp.int32),
            pltpu.VMEM((chunk, _DEG_W), jnp.float32),
            pltpu.VMEM_SHARED((N_PAD, _DEG_W), jnp.float32),
            pltpu.SemaphoreType.DMA,
        ],
        compiler_params=_SC_PARAMS,
    )
    def deg(ei_hbm, ones_hbm, zeros_hbm, out_hbm, dst_v, ones_v, acc_sh, sem):
        c = lax.axis_index("c")
        s = lax.axis_index("s")
        base = (c * NS + s) * N_PER_W
        _stage_indices(ei_hbm, 1, dst_v, base)
        pltpu.sync_copy(ones_hbm, ones_v)
        pltpu.sync_copy(zeros_hbm.at[pl.ds(s * stripe, stripe)],
                        acc_sh.at[pl.ds(s * stripe, stripe)])
        plsc.subcore_barrier()

        def fire(j, carry):
            pltpu.async_copy(
                ones_v, acc_sh.at[dst_v.at[pl.ds(j * chunk, chunk)]],
                sem, add=True)
            return carry

        def drain(j, carry):
            pltpu.make_async_copy(
                ones_v, acc_sh.at[dst_v.at[pl.ds(j * chunk, chunk)]],
                sem).wait()
            return carry

        lax.fori_loop(0, kb, fire, 0)
        lax.fori_loop(0, kb, drain, 0)
        plsc.subcore_barrier()
        pltpu.sync_copy(acc_sh.at[pl.ds(s * stripe, stripe)],
                        out_hbm.at[c].at[pl.ds(s * stripe, stripe)])

    return deg


@functools.lru_cache(maxsize=None)
def _agg_fn(d, chunk, nbuf):
    return _make_agg(d, chunk, nbuf)


@functools.lru_cache(maxsize=None)
def _deg_fn(chunk):
    return _make_deg(chunk)


def _agg_deg(*a):
    return _deg_fn(1024)(*a)


def _agg64(*a):
    return _agg_fn(L1, 512, 2)(*a)


def _agg32(*a):
    return _agg_fn(L2, 1024, 2)(*a)


_BLK = 2048
_GRID = N_PAD // _BLK


def _dinv_of(degp):
    return lax.rsqrt(degp[0] + degp[1] + 1.0)[:, None]


def _mm_body(x_ref, w_ref, out_ref):
    out_ref[...] = jnp.dot(x_ref[...], w_ref[...],
                           preferred_element_type=jnp.float32)


def _scale_body(h_ref, degp_ref, out_ref):
    out_ref[...] = _dinv_of(degp_ref[...]) * h_ref[...]


def _k2_body(p_ref, degp_ref, g1_ref, b_ref, w_ref, out_ref):
    dinv = _dinv_of(degp_ref[...])
    a = jnp.maximum(dinv * (p_ref[0] + p_ref[1] + g1_ref[...]) + b_ref[...],
                    0.0)
    out_ref[...] = dinv * jnp.dot(a, w_ref[...],
                                  preferred_element_type=jnp.float32)


def _k3_body(q_ref, degp_ref, g2_ref, b_ref, w_ref, b3_ref, out_ref):
    dinv = _dinv_of(degp_ref[...])
    a = jnp.maximum(dinv * (q_ref[0] + q_ref[1] + g2_ref[...]) + b_ref[...],
                    0.0)
    out_ref[...] = jnp.dot(a, w_ref[...],
                           preferred_element_type=jnp.float32) + b3_ref[...]


def _tc_matmul(x, w):
    return pl.pallas_call(
        _mm_body,
        grid=(_GRID,),
        in_specs=[
            pl.BlockSpec((_BLK, D_FEAT), lambda i: (i, 0)),
            pl.BlockSpec((D_FEAT, L1), lambda i: (0, 0)),
        ],
        out_specs=pl.BlockSpec((_BLK, L1), lambda i: (i, 0)),
        out_shape=jax.ShapeDtypeStruct((N_PAD, L1), jnp.float32),
    )(x, w)


def _tc_scale(h, degp):
    return pl.pallas_call(
        _scale_body,
        grid=(_GRID,),
        in_specs=[
            pl.BlockSpec((_BLK, L1), lambda i: (i, 0)),
            pl.BlockSpec((NC, _BLK), lambda i: (0, i)),
        ],
        out_specs=pl.BlockSpec((_BLK, L1), lambda i: (i, 0)),
        out_shape=jax.ShapeDtypeStruct((N_PAD, L1), jnp.float32),
    )(h, degp)


def _tc_layer2(p, degp, g1, b1, w2):
    return pl.pallas_call(
        _k2_body,
        grid=(_GRID,),
        in_specs=[
            pl.BlockSpec((NC, _BLK, L1), lambda i: (0, i, 0)),
            pl.BlockSpec((NC, _BLK), lambda i: (0, i)),
            pl.BlockSpec((_BLK, L1), lambda i: (i, 0)),
            pl.BlockSpec((1, L1), lambda i: (0, 0)),
            pl.BlockSpec((L1, L2), lambda i: (0, 0)),
        ],
        out_specs=pl.BlockSpec((_BLK, L2), lambda i: (i, 0)),
        out_shape=jax.ShapeDtypeStruct((N_PAD, L2), jnp.float32),
    )(p, degp, g1, b1, w2)


def _tc_head(q, degp, g2, b2, w3, b3):
    return pl.pallas_call(
        _k3_body,
        grid=(_GRID,),
        in_specs=[
            pl.BlockSpec((NC, _BLK, L2), lambda i: (0, i, 0)),
            pl.BlockSpec((NC, _BLK), lambda i: (0, i)),
            pl.BlockSpec((_BLK, L2), lambda i: (i, 0)),
            pl.BlockSpec((1, L2), lambda i: (0, 0)),
            pl.BlockSpec((L2, 1), lambda i: (0, 0)),
            pl.BlockSpec((1, 1), lambda i: (0, 0)),
        ],
        out_specs=pl.BlockSpec((_BLK, 1), lambda i: (i, 0)),
        out_shape=jax.ShapeDtypeStruct((N_PAD, 1), jnp.float32),
    )(q, degp, g2, b2, w3, b3)


def kernel(x, edge_index, W1, b1, W2, b2, W3, b3):
    ei = edge_index.astype(jnp.int32)

    x_pad = jnp.pad(x, ((0, N_PAD - N_NODES), (0, 0)))
    ones16 = jnp.ones((1024, _DEG_W), jnp.float32)
    zeros16 = jnp.zeros((N_PAD, _DEG_W), jnp.float32)
    zeros64 = jnp.zeros((N_PAD, L1), jnp.float32)
    zeros32 = jnp.zeros((N_PAD, L2), jnp.float32)

    degp = _agg_deg(ei, ones16, zeros16)[:, :, 0]
    h1 = _tc_matmul(x_pad, W1)

    g1 = _tc_scale(h1, degp)
    p = _agg64(g1, ei, zeros64)
    g2 = _tc_layer2(p, degp, g1, b1.reshape(1, L1), W2)
    q = _agg32(g2, ei, zeros32)
    out = _tc_head(q, degp, g2, b2.reshape(1, L2), W3, b3.reshape(1, 1))
    return out[:N_NODES]


# speedup vs baseline: 1.0971x; 1.0037x over previous
import functools

import jax
import jax.numpy as jnp
from jax import lax
from jax.experimental import pallas as pl
from jax.experimental.pallas import tpu as pltpu
from jax.experimental.pallas import tpu_sc as plsc

N_NODES = 10000
N_EDGES = 320000
D_FEAT = 128
L1 = 64
L2 = 32

NC = 2
NS = 16
NW = NC * NS
N_PAD = 10240
N_PER_W = N_EDGES // NW
PADT = 240
W_EDGES = N_PER_W + PADT
_DEG_W = 8

_SC_PARAMS = pltpu.CompilerParams(use_tc_tiling_on_sc=False)


def _mesh():
    return plsc.VectorSubcoreMesh(core_axis_name="c", subcore_axis_name="s",
                                  num_cores=NC, num_subcores=NS)


def _stage_indices(ei_hbm, row, idx_v, base):
    pltpu.sync_copy(ei_hbm.at[row].at[pl.ds(base, N_PER_W)],
                    idx_v.at[pl.ds(0, N_PER_W)])
    for t in range(PADT // 16):
        idx_v[pl.ds(N_PER_W + 16 * t, 16)] = (
            lax.iota(jnp.int32, 16) + (N_NODES + 16 * t))


def _make_agg(d, chunk, nbuf):
    kb = W_EDGES // chunk
    stripe = N_PAD // NS

    @functools.partial(
        pl.kernel,
        out_type=jax.ShapeDtypeStruct((NC, N_PAD, d), jnp.float32),
        mesh=_mesh(),
        scratch_types=[
            pltpu.VMEM((W_EDGES,), jnp.int32),
            pltpu.VMEM((W_EDGES,), jnp.int32),
            pltpu.VMEM((nbuf, chunk, d), jnp.float32),
            pltpu.VMEM_SHARED((N_PAD, d), jnp.float32),
            pltpu.SemaphoreType.DMA((nbuf,)),
        ],
        compiler_params=_SC_PARAMS,
    )
    def agg(table_hbm, ei_hbm, zeros_hbm, out_hbm,
            src_v, dst_v, rows_v, acc_sh, sems):
        c = lax.axis_index("c")
        s = lax.axis_index("s")
        base = (c * NS + s) * N_PER_W
        _stage_indices(ei_hbm, 0, src_v, base)
        _stage_indices(ei_hbm, 1, dst_v, base)
        pltpu.sync_copy(zeros_hbm.at[pl.ds(s * stripe, stripe)],
                        acc_sh.at[pl.ds(s * stripe, stripe)])
        plsc.subcore_barrier()

        def start_gather(j, b):
            pltpu.async_copy(
                table_hbm.at[src_v.at[pl.ds(j * chunk, chunk)]],
                rows_v.at[b], sems.at[b])

        for b in range(nbuf - 1):
            start_gather(b, b)

        def body(j, carry):
            b = lax.rem(j, nbuf)
            f = j + nbuf - 1

            @pl.when(f < kb)
            def _():
                start_gather(f, lax.rem(f, nbuf))

            pltpu.make_async_copy(
                table_hbm.at[src_v.at[pl.ds(j * chunk, chunk)]],
                rows_v.at[b], sems.at[b]).wait()
            pltpu.sync_copy(rows_v.at[b],
                            acc_sh.at[dst_v.at[pl.ds(j * chunk, chunk)]],
                            add=True)
            return carry

        lax.fori_loop(0, kb, body, 0)
        plsc.subcore_barrier()
        pltpu.sync_copy(acc_sh.at[pl.ds(s * stripe, stripe)],
                        out_hbm.at[c].at[pl.ds(s * stripe, stripe)])

    return agg


def _make_deg(chunk):
    kb = W_EDGES // chunk
    stripe = N_PAD // NS

    @functools.partial(
        pl.kernel,
        out_type=jax.ShapeDtypeStruct((NC, N_PAD, _DEG_W), jnp.float32),
        mesh=_mesh(),
        scratch_types=[
            pltpu.VMEM((W_EDGES,), jnp.int32),
            pltpu.VMEM((chunk, _DEG_W), jnp.float32),
            pltpu.VMEM_SHARED((N_PAD, _DEG_W), jnp.float32),
            pltpu.SemaphoreType.DMA,
        ],
        compiler_params=_SC_PARAMS,
    )
    def deg(ei_hbm, ones_hbm, zeros_hbm, out_hbm, dst_v, ones_v, acc_sh, sem):
        c = lax.axis_index("c")
        s = lax.axis_index("s")
        base = (c * NS + s) * N_PER_W
        _stage_indices(ei_hbm, 1, dst_v, base)
        pltpu.sync_copy(ones_hbm, ones_v)
        pltpu.sync_copy(zeros_hbm.at[pl.ds(s * stripe, stripe)],
                        acc_sh.at[pl.ds(s * stripe, stripe)])
        plsc.subcore_barrier()

        def fire(j, carry):
            pltpu.async_copy(
                ones_v, acc_sh.at[dst_v.at[pl.ds(j * chunk, chunk)]],
                sem, add=True)
            return carry

        def drain(j, carry):
            pltpu.make_async_copy(
                ones_v, acc_sh.at[dst_v.at[pl.ds(j * chunk, chunk)]],
                sem).wait()
            return carry

        lax.fori_loop(0, kb, fire, 0)
        lax.fori_loop(0, kb, drain, 0)
        plsc.subcore_barrier()
        pltpu.sync_copy(acc_sh.at[pl.ds(s * stripe, stripe)],
                        out_hbm.at[c].at[pl.ds(s * stripe, stripe)])

    return deg


@functools.lru_cache(maxsize=None)
def _agg_fn(d, chunk, nbuf):
    return _make_agg(d, chunk, nbuf)


@functools.lru_cache(maxsize=None)
def _deg_fn(chunk):
    return _make_deg(chunk)


def _agg_deg(*a):
    return _deg_fn(1024)(*a)


def _agg64(*a):
    return _agg_fn(L1, 512, 2)(*a)


def _agg32(*a):
    return _agg_fn(L2, 1024, 2)(*a)


_BLK = 2048
_GRID = N_PAD // _BLK


def _dinv_of(deg):
    return lax.rsqrt(deg[0])[:, None]


def _mm_body(x_ref, w_ref, out_ref):
    out_ref[...] = jnp.dot(x_ref[...], w_ref[...],
                           preferred_element_type=jnp.float32)


def _scale_body(h_ref, degp_ref, out_ref):
    out_ref[...] = _dinv_of(degp_ref[...]) * h_ref[...]


def _k2_body(p_ref, degp_ref, g1_ref, b_ref, w_ref, out_ref):
    dinv = _dinv_of(degp_ref[...])
    a = jnp.maximum(dinv * (p_ref[0] + p_ref[1] + g1_ref[...]) + b_ref[...],
                    0.0)
    out_ref[...] = dinv * jnp.dot(a, w_ref[...],
                                  preferred_element_type=jnp.float32)


def _k3_body(q_ref, degp_ref, g2_ref, b_ref, w_ref, b3_ref, out_ref):
    dinv = _dinv_of(degp_ref[...])
    a = jnp.maximum(dinv * (q_ref[0] + q_ref[1] + g2_ref[...]) + b_ref[...],
                    0.0)
    out_ref[...] = jnp.dot(a, w_ref[...],
                           preferred_element_type=jnp.float32) + b3_ref[...]


def _tc_matmul(x, w):
    return pl.pallas_call(
        _mm_body,
        grid=(_GRID,),
        in_specs=[
            pl.BlockSpec((_BLK, D_FEAT), lambda i: (i, 0)),
            pl.BlockSpec((D_FEAT, L1), lambda i: (0, 0)),
        ],
        out_specs=pl.BlockSpec((_BLK, L1), lambda i: (i, 0)),
        out_shape=jax.ShapeDtypeStruct((N_PAD, L1), jnp.float32),
    )(x, w)


def _tc_scale(h, degp):
    return pl.pallas_call(
        _scale_body,
        grid=(_GRID,),
        in_specs=[
            pl.BlockSpec((_BLK, L1), lambda i: (i, 0)),
            pl.BlockSpec((1, _BLK), lambda i: (0, i)),
        ],
        out_specs=pl.BlockSpec((_BLK, L1), lambda i: (i, 0)),
        out_shape=jax.ShapeDtypeStruct((N_PAD, L1), jnp.float32),
    )(h, degp)


def _tc_layer2(p, degp, g1, b1, w2):
    return pl.pallas_call(
        _k2_body,
        grid=(_GRID,),
        in_specs=[
            pl.BlockSpec((NC, _BLK, L1), lambda i: (0, i, 0)),
            pl.BlockSpec((1, _BLK), lambda i: (0, i)),
            pl.BlockSpec((_BLK, L1), lambda i: (i, 0)),
            pl.BlockSpec((1, L1), lambda i: (0, 0)),
            pl.BlockSpec((L1, L2), lambda i: (0, 0)),
        ],
        out_specs=pl.BlockSpec((_BLK, L2), lambda i: (i, 0)),
        out_shape=jax.ShapeDtypeStruct((N_PAD, L2), jnp.float32),
    )(p, degp, g1, b1, w2)


def _tc_head(q, degp, g2, b2, w3, b3):
    return pl.pallas_call(
        _k3_body,
        grid=(_GRID,),
        in_specs=[
            pl.BlockSpec((NC, _BLK, L2), lambda i: (0, i, 0)),
            pl.BlockSpec((1, _BLK), lambda i: (0, i)),
            pl.BlockSpec((_BLK, L2), lambda i: (i, 0)),
            pl.BlockSpec((1, L2), lambda i: (0, 0)),
            pl.BlockSpec((L2, 1), lambda i: (0, 0)),
            pl.BlockSpec((1, 1), lambda i: (0, 0)),
        ],
        out_specs=pl.BlockSpec((_BLK, 1), lambda i: (i, 0)),
        out_shape=jax.ShapeDtypeStruct((N_PAD, 1), jnp.float32),
    )(q, degp, g2, b2, w3, b3)


def kernel(x, edge_index, W1, b1, W2, b2, W3, b3):
    ei = edge_index.astype(jnp.int32)

    x_pad = jnp.pad(x, ((0, N_PAD - N_NODES), (0, 0)))
    ones16 = jnp.ones((1024, _DEG_W), jnp.float32)
    zeros16 = jnp.zeros((N_PAD, _DEG_W), jnp.float32)
    zeros64 = jnp.zeros((N_PAD, L1), jnp.float32)
    zeros32 = jnp.zeros((N_PAD, L2), jnp.float32)

    dp = _agg_deg(ei, ones16, zeros16)
    h1 = _tc_matmul(x_pad, W1)
    degp = (dp[0, :, 0] + dp[1, :, 0] + 1.0)[None, :]

    g1 = _tc_scale(h1, degp)
    p = _agg64(g1, ei, zeros64)
    g2 = _tc_layer2(p, degp, g1, b1.reshape(1, L1), W2)
    q = _agg32(g2, ei, zeros32)
    out = _tc_head(q, degp, g2, b2.reshape(1, L2), W3, b3.reshape(1, 1))
    return out[:N_NODES]
